# baseline (speedup 1.0000x reference)
import functools

import jax
import jax.numpy as jnp
from jax import lax
from jax.experimental import pallas as pl
from jax.experimental.pallas import tpu as pltpu
from jax.experimental.pallas import tpu_sc as plsc

EMB = 128
TOTAL_ROWS = 4096 * 200


def _make_sc_lookup(total_rows, emb):
    info = plsc.get_sparse_core_info()
    nc, ns = info.num_cores, info.num_subcores
    nw = nc * ns
    rows_per_w = total_rows // nw
    CH = 128
    n_ch = rows_per_w // CH

    mesh = plsc.VectorSubcoreMesh(core_axis_name="c", subcore_axis_name="s")

    @functools.partial(
        pl.kernel,
        mesh=mesh,
        out_type=jax.ShapeDtypeStruct((total_rows, emb), jnp.float32),
        scratch_types=[
            pltpu.VMEM((CH,), jnp.int32),
            pltpu.VMEM((CH, emb), jnp.float32),
            pltpu.SemaphoreType.DMA,
        ],
    )
    def k(idx_hbm, table_hbm, out_hbm, idx_v, rows_v, sem):
        wid = lax.axis_index("s") * nc + lax.axis_index("c")
        base = wid * rows_per_w

        def body(i, carry):
            b = base + i * CH
            pltpu.sync_copy(idx_hbm.at[pl.ds(b, CH)], idx_v)
            pltpu.async_copy(table_hbm.at[idx_v], rows_v, sem).wait()
            pltpu.sync_copy(rows_v, out_hbm.at[pl.ds(b, CH)])
            return carry

        lax.fori_loop(0, n_ch, body, 0)

    return k


_sc_lookup = _make_sc_lookup(TOTAL_ROWS, EMB)


def kernel(indices, table):
    B, T = indices.shape
    flat = indices.reshape(B * T).astype(jnp.int32)
    out = _sc_lookup(flat, table)
    return out.reshape(B, T, EMB)


# speedup vs baseline: 52.6726x; 52.6726x over previous
import functools

import jax
import jax.numpy as jnp
from jax import lax
from jax.experimental import pallas as pl
from jax.experimental.pallas import tpu as pltpu
from jax.experimental.pallas import tpu_sc as plsc

EMB = 128
TOTAL_ROWS = 4096 * 200
CH = 128
NB = 4


def _make_sc_lookup(total_rows, emb):
    info = plsc.get_sparse_core_info()
    nc, ns = info.num_cores, info.num_subcores
    nw = nc * ns
    rows_per_w = total_rows // nw
    n_steps = rows_per_w // CH
    n_groups = n_steps // NB

    mesh = plsc.VectorSubcoreMesh(core_axis_name="c", subcore_axis_name="s")

    @functools.partial(
        pl.kernel,
        mesh=mesh,
        out_type=jax.ShapeDtypeStruct((total_rows, emb), jnp.float32),
        scratch_types=[
            pltpu.VMEM_SHARED((3, emb), jnp.float32),
            pltpu.VMEM((n_steps, CH), jnp.int32),
            pltpu.VMEM((NB, CH, emb), jnp.float32),
            pltpu.SemaphoreType.DMA,
        ] + [pltpu.SemaphoreType.DMA] * NB,
    )
    def k(idx_hbm, table_hbm, out_hbm, table_v, idx_v, rows_v, gsem, *wsems):
        wid = lax.axis_index("s") * nc + lax.axis_index("c")
        base = wid * rows_per_w
        @pl.when(lax.axis_index("s") == 0)
        def _():
            pltpu.sync_copy(table_hbm, table_v)

        pltpu.sync_copy(idx_hbm.at[pl.ds(wid * n_steps, n_steps)], idx_v)
        plsc.subcore_barrier()

        def group(g, carry):
            for b in range(NB):
                s = g * NB + b

                @pl.when(g > 0)
                def _():
                    pltpu.make_async_copy(
                        rows_v.at[b], out_hbm.at[pl.ds(base, CH)], wsems[b]
                    ).wait()

                pltpu.async_copy(table_v.at[idx_v.at[s]], rows_v.at[b], gsem).wait()
                pltpu.async_copy(
                    rows_v.at[b], out_hbm.at[pl.ds(base + s * CH, CH)], wsems[b]
                )
            return carry

        lax.fori_loop(0, n_groups, group, 0)
        for b in range(NB):
            pltpu.make_async_copy(
                rows_v.at[b], out_hbm.at[pl.ds(base, CH)], wsems[b]
            ).wait()

    return k


_sc_lookup = _make_sc_lookup(TOTAL_ROWS, EMB)


def kernel(indices, table):
    B, T = indices.shape
    flat = indices.reshape(B * T).astype(jnp.int32)
    idx2d = flat.reshape(TOTAL_ROWS // CH, CH)
    out = _sc_lookup(idx2d, table)
    return out.reshape(B, T, EMB)
